# baseline (speedup 1.0000x reference)
import jax
import jax.numpy as jnp
from jax import lax
from jax.experimental import pallas as pl
from jax.experimental.pallas import tpu as pltpu
from jax.experimental.pallas import tpu_sc as plsc

_N = 10000
_E = 320000
_F = 128
_WEXT = 144
_NC, _NS, _L = 2, 16, 16
_NW = _NC * _NS
_CH = 64
_NCH = 159
_EPW = _NCH * _CH
_EPAD = _NW * _EPW
_BN = 1000
_NPAD = 10112
_RPS = _NPAD // _NS


def _prep_body(x_ref, w_ref, a_ref, hext_ref, spk_ref):
    x = x_ref[...]
    w = w_ref[...]
    h = jnp.dot(x, w, preferred_element_type=jnp.float32)
    ones = jnp.ones((x.shape[0], 1), jnp.float32)
    pad = jnp.zeros((x.shape[0], _WEXT - _F - 1), jnp.float32)
    hext_ref[...] = jnp.concatenate([h, ones, pad], axis=1)
    a2 = a_ref[...].reshape(2, _F)
    s12 = lax.dot_general(h, a2, (((1,), (1,)), ((), ())),
                          preferred_element_type=jnp.float32)
    u = lax.bitcast_convert_type(s12, jnp.uint32)
    packed = (u[:, 0:1] & jnp.uint32(0xFFFF0000)) | (u[:, 1:2] >> 16)
    spk_ref[...] = lax.bitcast_convert_type(packed, jnp.float32)


def _prep(x, w, a):
    return pl.pallas_call(
        _prep_body,
        grid=(_N // _BN,),
        in_specs=[
            pl.BlockSpec((_BN, _F), lambda i: (i, 0)),
            pl.BlockSpec((_F, _F), lambda i: (0, 0)),
            pl.BlockSpec((1, 2 * _F), lambda i: (0, 0)),
        ],
        out_specs=[
            pl.BlockSpec((_BN, _WEXT), lambda i: (i, 0)),
            pl.BlockSpec((_BN, 1), lambda i: (i, 0)),
        ],
        out_shape=[
            jax.ShapeDtypeStruct((_N, _WEXT), jnp.float32),
            jax.ShapeDtypeStruct((_N, 1), jnp.float32),
        ],
    )(x, w, a)


def _sc_body(src_hbm, dst_hbm, hext_hbm, spk_hbm, part_hbm,
             si0, di0, si1, di1, si2, di2, sc0, sc1, sc2, spk_v,
             r0, r1, r2, acc_sh,
             sg0, sg1, sg2, ss0, ss1, ss2, sj0, sj1, sj2):
    cid = lax.axis_index("c")
    sid = lax.axis_index("s")
    wid = sid * _NC + cid
    sidx = (si0, si1, si2)
    didx = (di0, di1, di2)
    sidxs = (sc0, sc1, sc2)
    rows = (r0, r1, r2)
    semg = (sg0, sg1, sg2)
    sems = (ss0, ss1, ss2)
    semi = (sj0, sj1, sj2)
    mask_hi = jnp.full((_L,), -65536, jnp.int32)

    pltpu.async_copy(spk_hbm, spk_v, sg0).wait()

    @pl.loop(0, _CH)
    def _zero_rows(r):
        for j in range(_WEXT // _L):
            r0[r, pl.ds(j * _L, _L)] = jnp.zeros((_L,), jnp.float32)

    rowbase = sid * _RPS
    for z in range(_RPS // _CH):
        pltpu.sync_copy(r0, acc_sh.at[pl.ds(rowbase + z * _CH, _CH)])
    pltpu.sync_copy(r0.at[pl.ds(0, _RPS - _RPS // _CH * _CH)],
                    acc_sh.at[pl.ds(rowbase + _RPS // _CH * _CH,
                                    _RPS - _RPS // _CH * _CH)])
    plsc.subcore_barrier()

    def issue_idx(k, b):
        pltpu.async_copy(src_hbm.at[wid, k], sidx[b].at[0], semi[b])
        pltpu.async_copy(dst_hbm.at[wid, k], didx[b].at[0], semi[b])

    def wait_idx(k, b):
        pltpu.make_async_copy(src_hbm.at[wid, k], sidx[b].at[0],
                              semi[b]).wait()
        pltpu.make_async_copy(dst_hbm.at[wid, k], didx[b].at[0],
                              semi[b]).wait()

    def issue_gather(k, b):
        pltpu.async_copy(hext_hbm.at[didx[b].at[0]], rows[b], semg[b])

    def wait_gather(k, b):
        pltpu.make_async_copy(hext_hbm.at[didx[b].at[0]], rows[b],
                              semg[b]).wait()

    def issue_scatter(k, b):
        pltpu.async_copy(rows[b], acc_sh.at[sidxs[b].at[0]], sems[b],
                         add=True)

    def wait_scatter(k, b):
        pltpu.make_async_copy(rows[b], acc_sh.at[sidxs[b].at[0]],
                              sems[b]).wait()

    def compute(k, b):
        row_v = rows[b]
        for g in range(_CH // _L):
            s16 = sidx[b][0, pl.ds(g * _L, _L)]
            d16 = didx[b][0, pl.ds(g * _L, _L)]
            sidxs[b][0, pl.ds(g * _L, _L)] = s16
            v1 = plsc.load_gather(spk_v, [s16])
            v2 = plsc.load_gather(spk_v, [d16])
            s1 = plsc.bitcast(plsc.bitcast(v1, jnp.int32) & mask_hi,
                              jnp.float32)
            s2 = plsc.bitcast(plsc.bitcast(v2, jnp.int32) << 16, jnp.float32)
            t = s1 + s2
            e16 = jnp.exp(jnp.where(t > 0, -t, -0.2 * t))
            for i in range(_L):
                es = e16[i]
                row = g * _L + i
                for j in range(_WEXT // _L):
                    sl = pl.ds(j * _L, _L)
                    row_v[row, sl] = row_v[row, sl] * es

    pltpu.sync_copy(src_hbm.at[wid, 0], si0.at[0])
    pltpu.sync_copy(dst_hbm.at[wid, 0], di0.at[0])
    issue_gather(0, 0)
    issue_idx(1, 1)
    issue_idx(2, 2)

    @pl.loop(0, _NCH, step=3)
    def _triple(c):
        for j in range(3):
            cj = c + j
            b = j
            bn = (j + 1) % 3
            with jax.named_scope(f"sub{j}"):
                @pl.when(cj + 1 < _NCH)
                def _(cj=cj, bn=bn):
                    wait_idx(cj + 1, bn)

                @pl.when(cj >= 2)
                def _(cj=cj, bn=bn):
                    wait_scatter(cj - 2, bn)

                @pl.when(cj + 1 < _NCH)
                def _(cj=cj, bn=bn):
                    issue_gather(cj + 1, bn)

                wait_gather(cj, b)
                compute(cj, b)
                issue_scatter(cj, b)

                @pl.when(cj + 3 < _NCH)
                def _(cj=cj, b=b):
                    issue_idx(cj + 3, b)

    wait_scatter(_NCH - 2, (_NCH - 2) % 3)
    wait_scatter(_NCH - 1, (_NCH - 1) % 3)

    plsc.subcore_barrier()
    for z in range(_RPS // 158):
        r0w = rowbase + z * 158
        pltpu.sync_copy(acc_sh.at[pl.ds(r0w, 158)],
                        part_hbm.at[cid, pl.ds(r0w, 158)])


def _sc_accumulate(srcd, dstd, hext, spk):
    mesh = plsc.VectorSubcoreMesh(core_axis_name="c", subcore_axis_name="s")
    kern = pl.kernel(
        _sc_body,
        out_type=jax.ShapeDtypeStruct((_NC, _NPAD, _WEXT), jnp.float32),
        mesh=mesh,
        scratch_types=[
            pltpu.VMEM((1, _CH), jnp.int32),
            pltpu.VMEM((1, _CH), jnp.int32),
            pltpu.VMEM((1, _CH), jnp.int32),
            pltpu.VMEM((1, _CH), jnp.int32),
            pltpu.VMEM((1, _CH), jnp.int32),
            pltpu.VMEM((1, _CH), jnp.int32),
            pltpu.VMEM((1, _CH), jnp.int32),
            pltpu.VMEM((1, _CH), jnp.int32),
            pltpu.VMEM((1, _CH), jnp.int32),
            pltpu.VMEM((_NPAD,), jnp.float32),
            pltpu.VMEM((_CH, _WEXT), jnp.float32),
            pltpu.VMEM((_CH, _WEXT), jnp.float32),
            pltpu.VMEM((_CH, _WEXT), jnp.float32),
            pltpu.VMEM_SHARED((_NPAD, _WEXT), jnp.float32),
            pltpu.SemaphoreType.DMA,
            pltpu.SemaphoreType.DMA,
            pltpu.SemaphoreType.DMA,
            pltpu.SemaphoreType.DMA,
            pltpu.SemaphoreType.DMA,
            pltpu.SemaphoreType.DMA,
            pltpu.SemaphoreType.DMA,
            pltpu.SemaphoreType.DMA,
            pltpu.SemaphoreType.DMA,
        ],
        compiler_params=pltpu.CompilerParams(use_tc_tiling_on_sc=False,
                                             needs_layout_passes=False),
    )
    return kern(srcd, dstd, hext, spk)


def _final_body(part_ref, out_ref):
    p = part_ref[0] + part_ref[1]
    r = p[:, 0:_F] / p[:, _F:_F + 1]
    out_ref[...] = jnp.where(r > 0, r, jnp.exp(jnp.minimum(r, 0.0)) - 1.0)


def _final(part):
    return pl.pallas_call(
        _final_body,
        grid=(_N // _BN,),
        in_specs=[pl.BlockSpec((_NC, _BN, _WEXT), lambda i: (0, i, 0))],
        out_specs=pl.BlockSpec((_BN, _F), lambda i: (i, 0)),
        out_shape=jax.ShapeDtypeStruct((_N, _F), jnp.float32),
    )(part)


def kernel(input, edge, W, a):
    hext, spk = _prep(input, W, a)
    spk = jnp.concatenate(
        [spk.reshape(_N), jnp.zeros((_NPAD - _N,), jnp.float32)])
    srcp = jnp.concatenate(
        [edge[0], jnp.full((_EPAD - _E,), _NPAD - 1, jnp.int32)])
    dstp = jnp.concatenate(
        [edge[1], jnp.zeros((_EPAD - _E,), jnp.int32)])
    srcd = srcp.reshape(_NW, _NCH, _CH)
    dstd = dstp.reshape(_NW, _NCH, _CH)
    part = _sc_accumulate(srcd, dstd, hext, spk)
    return _final(part)


# speedup vs baseline: 1.3600x; 1.3600x over previous
import jax
import jax.numpy as jnp
from jax import lax
from jax.experimental import pallas as pl
from jax.experimental.pallas import tpu as pltpu
from jax.experimental.pallas import tpu_sc as plsc

_N = 10000
_E = 320000
_F = 128
_WEXT = 144
_NC, _NS, _L = 2, 16, 16
_NW = _NC * _NS
_EPW = _E // _NW
_CH = 80
_NCH = _EPW // _CH
_BN = 1000
_NPAD = 10240
_RPS = _NPAD // _NS


def _prep_body(x_ref, w_ref, a_ref, hext_ref, spk_ref):
    x = x_ref[...]
    w = w_ref[...]
    h = jnp.dot(x, w, preferred_element_type=jnp.float32)
    ones = jnp.ones((x.shape[0], 1), jnp.float32)
    pad = jnp.zeros((x.shape[0], _WEXT - _F - 1), jnp.float32)
    hext_ref[...] = jnp.concatenate([h, ones, pad], axis=1)
    a2 = a_ref[...].reshape(2, _F)
    s12 = lax.dot_general(h, a2, (((1,), (1,)), ((), ())),
                          preferred_element_type=jnp.float32)
    u = lax.bitcast_convert_type(s12, jnp.uint32)
    packed = (u[:, 0:1] & jnp.uint32(0xFFFF0000)) | (u[:, 1:2] >> 16)
    spk_ref[...] = lax.bitcast_convert_type(packed, jnp.float32)


def _prep(x, w, a):
    return pl.pallas_call(
        _prep_body,
        grid=(_N // _BN,),
        in_specs=[
            pl.BlockSpec((_BN, _F), lambda i: (i, 0)),
            pl.BlockSpec((_F, _F), lambda i: (0, 0)),
            pl.BlockSpec((1, 2 * _F), lambda i: (0, 0)),
        ],
        out_specs=[
            pl.BlockSpec((_BN, _WEXT), lambda i: (i, 0)),
            pl.BlockSpec((_BN, 1), lambda i: (i, 0)),
        ],
        out_shape=[
            jax.ShapeDtypeStruct((_N, _WEXT), jnp.float32),
            jax.ShapeDtypeStruct((_N, 1), jnp.float32),
        ],
    )(x, w, a)


def _idx_copy(src_hbm, dst_hbm, sidx_v, didx_v, wid, k, sem):
    a = pltpu.make_async_copy(src_hbm.at[wid, k], sidx_v.at[0], sem)
    b = pltpu.make_async_copy(dst_hbm.at[wid, k], didx_v.at[0], sem)
    return a, b


def _sc_body(src_hbm, dst_hbm, hext_hbm, spk_hbm, part_hbm,
             sidx0_v, didx0_v, sidx1_v, didx1_v, spk_v, row0_v, row1_v,
             acc_sh, semg0, semg1, semi0, semi1):
    cid = lax.axis_index("c")
    sid = lax.axis_index("s")
    wid = sid * _NC + cid
    sidx = (sidx0_v, sidx1_v)
    didx = (didx0_v, didx1_v)
    rows = (row0_v, row1_v)
    semg = (semg0, semg1)
    semi = (semi0, semi1)
    mask_hi = jnp.full((_L,), -65536, jnp.int32)

    pltpu.async_copy(spk_hbm, spk_v, semg0).wait()

    @pl.loop(0, _CH)
    def _zero_rows(r):
        for j in range(_WEXT // _L):
            row0_v[r, pl.ds(j * _L, _L)] = jnp.zeros((_L,), jnp.float32)

    row0 = sid * _RPS
    for z in range(_RPS // _CH):
        pltpu.sync_copy(row0_v, acc_sh.at[pl.ds(row0 + z * _CH, _CH)])
    plsc.subcore_barrier()

    def issue_gather(k, b):
        return pltpu.async_copy(hext_hbm.at[didx[b].at[0]], rows[b], semg[b])

    def wait_gather(k, b):
        pltpu.make_async_copy(hext_hbm.at[didx[b].at[0]], rows[b],
                              semg[b]).wait()

    def issue_idx(k, b):
        for d in _idx_copy(src_hbm, dst_hbm, sidx[b], didx[b], wid, k,
                           semi[b]):
            d.start()

    def wait_idx(k, b):
        for d in _idx_copy(src_hbm, dst_hbm, sidx[b], didx[b], wid, k,
                           semi[b]):
            d.wait()

    def compute_scatter(k, b):
        row_v = rows[b]
        for g in range(_CH // _L):
            s16 = sidx[b][0, pl.ds(g * _L, _L)]
            d16 = didx[b][0, pl.ds(g * _L, _L)]
            v1 = plsc.load_gather(spk_v, [s16])
            v2 = plsc.load_gather(spk_v, [d16])
            s1 = plsc.bitcast(plsc.bitcast(v1, jnp.int32) & mask_hi,
                              jnp.float32)
            s2 = plsc.bitcast(plsc.bitcast(v2, jnp.int32) << 16, jnp.float32)
            t = s1 + s2
            e16 = jnp.exp(jnp.where(t > 0, -t, -0.2 * t))
            for i in range(_L):
                es = e16[i]
                row = g * _L + i
                esv = jnp.full((_L,), es, jnp.float32)
                for j in range(_F // _L):
                    sl = pl.ds(j * _L, _L)
                    row_v[row, sl] = row_v[row, sl] * es
                row_v[row, pl.ds(_F, _L)] = esv
        pltpu.sync_copy(row_v, acc_sh.at[sidx[b].at[0]], add=True)

    pltpu.sync_copy(src_hbm.at[wid, 0], sidx0_v.at[0])
    pltpu.sync_copy(dst_hbm.at[wid, 0], didx0_v.at[0])
    issue_gather(0, 0)
    issue_idx(1, 1)

    @pl.loop(0, _NCH - 1, step=2)
    def _pair(k):
        wait_idx(k + 1, 1)
        issue_gather(k + 1, 1)
        wait_gather(k, 0)
        compute_scatter(k, 0)
        issue_idx(k + 2, 0)
        wait_idx(k + 2, 0)
        issue_gather(k + 2, 0)
        wait_gather(k + 1, 1)
        compute_scatter(k + 1, 1)

        @pl.when(k < _NCH - 3)
        def _():
            issue_idx(k + 3, 1)

    wait_gather(_NCH - 1, 0)
    compute_scatter(_NCH - 1, 0)

    plsc.subcore_barrier()
    for z in range(_RPS // _CH):
        r0 = row0 + z * _CH
        pltpu.sync_copy(acc_sh.at[pl.ds(r0, _CH)],
                        part_hbm.at[cid, pl.ds(r0, _CH)])


def _sc_accumulate(srcd, dstd, hext, spk):
    mesh = plsc.VectorSubcoreMesh(core_axis_name="c", subcore_axis_name="s")
    kern = pl.kernel(
        _sc_body,
        out_type=jax.ShapeDtypeStruct((_NC, _NPAD, _WEXT), jnp.float32),
        mesh=mesh,
        scratch_types=[
            pltpu.VMEM((1, _CH), jnp.int32),
            pltpu.VMEM((1, _CH), jnp.int32),
            pltpu.VMEM((1, _CH), jnp.int32),
            pltpu.VMEM((1, _CH), jnp.int32),
            pltpu.VMEM((_N,), jnp.float32),
            pltpu.VMEM((_CH, _WEXT), jnp.float32),
            pltpu.VMEM((_CH, _WEXT), jnp.float32),
            pltpu.VMEM_SHARED((_NPAD, _WEXT), jnp.float32),
            pltpu.SemaphoreType.DMA,
            pltpu.SemaphoreType.DMA,
            pltpu.SemaphoreType.DMA,
            pltpu.SemaphoreType.DMA,
        ],
        compiler_params=pltpu.CompilerParams(use_tc_tiling_on_sc=False,
                                             needs_layout_passes=False),
    )
    return kern(srcd, dstd, hext, spk)


def _final_body(part_ref, out_ref):
    p = part_ref[0] + part_ref[1]
    r = p[:, 0:_F] / p[:, _F:_F + 1]
    out_ref[...] = jnp.where(r > 0, r, jnp.exp(jnp.minimum(r, 0.0)) - 1.0)


def _final(part):
    return pl.pallas_call(
        _final_body,
        grid=(_N // _BN,),
        in_specs=[pl.BlockSpec((_NC, _BN, _WEXT), lambda i: (0, i, 0))],
        out_specs=pl.BlockSpec((_BN, _F), lambda i: (i, 0)),
        out_shape=jax.ShapeDtypeStruct((_N, _F), jnp.float32),
    )(part)


def kernel(input, edge, W, a):
    hext, spk = _prep(input, W, a)
    spk = spk.reshape(_N)
    srcd = edge[0].reshape(_NW, _NCH, _CH)
    dstd = edge[1].reshape(_NW, _NCH, _CH)
    part = _sc_accumulate(srcd, dstd, hext, spk)
    return _final(part)


# speedup vs baseline: 1.4106x; 1.0372x over previous
import jax
import jax.numpy as jnp
from jax import lax
from jax.experimental import pallas as pl
from jax.experimental.pallas import tpu as pltpu
from jax.experimental.pallas import tpu_sc as plsc

_N = 10000
_E = 320000
_F = 128
_WEXT = 144
_NC, _NS, _L = 2, 16, 16
_NW = _NC * _NS
_EPW = _E // _NW
_CH = 80
_NCH = _EPW // _CH
_BN = 1000
_NPAD = 10240
_RPS = _NPAD // _NS


def _prep_body(x_ref, w_ref, a_ref, hw_ref, spk_ref):
    x = x_ref[...]
    w = w_ref[...]
    h = jnp.dot(x, w, preferred_element_type=jnp.float32)
    u = lax.bitcast_convert_type(h, jnp.uint32)
    parts = []
    for j in range(_F // 32):
        lo = u[:, 32 * j:32 * j + 16] >> 16
        hi = u[:, 32 * j + 16:32 * j + 32] & jnp.uint32(0xFFFF0000)
        parts.append(hi | lo)
    hw_ref[...] = jnp.concatenate(parts, axis=1)
    a2 = a_ref[...].reshape(2, _F)
    s12 = lax.dot_general(h, a2, (((1,), (1,)), ((), ())),
                          preferred_element_type=jnp.float32)
    up = lax.bitcast_convert_type(s12, jnp.uint32)
    packed = (up[:, 0:1] & jnp.uint32(0xFFFF0000)) | (up[:, 1:2] >> 16)
    spk_ref[...] = lax.bitcast_convert_type(packed, jnp.float32)


def _prep(x, w, a):
    return pl.pallas_call(
        _prep_body,
        grid=(_N // _BN,),
        in_specs=[
            pl.BlockSpec((_BN, _F), lambda i: (i, 0)),
            pl.BlockSpec((_F, _F), lambda i: (0, 0)),
            pl.BlockSpec((1, 2 * _F), lambda i: (0, 0)),
        ],
        out_specs=[
            pl.BlockSpec((_BN, _F // 2), lambda i: (i, 0)),
            pl.BlockSpec((_BN, 1), lambda i: (i, 0)),
        ],
        out_shape=[
            jax.ShapeDtypeStruct((_N, _F // 2), jnp.uint32),
            jax.ShapeDtypeStruct((_N, 1), jnp.float32),
        ],
    )(x, w, a)


def _idx_copy(src_hbm, dst_hbm, sidx_v, didx_v, wid, k, sem):
    a = pltpu.make_async_copy(src_hbm.at[wid, k], sidx_v.at[0], sem)
    b = pltpu.make_async_copy(dst_hbm.at[wid, k], didx_v.at[0], sem)
    return a, b


def _sc_body(src_hbm, dst_hbm, hw_hbm, spk_hbm, part_hbm,
             sidx0_v, didx0_v, sidx1_v, didx1_v, spk_v, rb0, rb1, fb_v,
             acc_sh, semg0, semg1, semi0, semi1):
    cid = lax.axis_index("c")
    sid = lax.axis_index("s")
    wid = sid * _NC + cid
    sidx = (sidx0_v, sidx1_v)
    didx = (didx0_v, didx1_v)
    rowb = (rb0, rb1)
    semg = (semg0, semg1)
    semi = (semi0, semi1)
    mask_hi = jnp.full((_L,), -65536, jnp.int32)
    umask_hi = jnp.full((_L,), 0xFFFF0000, jnp.uint32)

    pltpu.async_copy(spk_hbm, spk_v, semg0).wait()

    @pl.loop(0, _CH)
    def _zero_rows(r):
        for j in range(_WEXT // _L):
            fb_v[r, pl.ds(j * _L, _L)] = jnp.zeros((_L,), jnp.float32)

    row0 = sid * _RPS
    for z in range(_RPS // _CH):
        pltpu.sync_copy(fb_v, acc_sh.at[pl.ds(row0 + z * _CH, _CH)])
    plsc.subcore_barrier()

    def issue_gather(k, b):
        return pltpu.async_copy(hw_hbm.at[didx[b].at[0]], rowb[b], semg[b])

    def wait_gather(k, b):
        pltpu.make_async_copy(hw_hbm.at[didx[b].at[0]], rowb[b],
                              semg[b]).wait()

    def issue_idx(k, b):
        pltpu.async_copy(src_hbm.at[wid, k], sidx[b].at[0], semi[b])
        pltpu.async_copy(dst_hbm.at[wid, k], didx[b].at[0], semi[b])

    def wait_idx(k, b):
        pltpu.make_async_copy(src_hbm.at[wid, k], sidx[b].at[0],
                              semi[b]).wait()
        pltpu.make_async_copy(dst_hbm.at[wid, k], didx[b].at[0],
                              semi[b]).wait()

    def compute_scatter(k, b):
        rb = rowb[b]
        for g in range(_CH // _L):
            s16 = sidx[b][0, pl.ds(g * _L, _L)]
            d16 = didx[b][0, pl.ds(g * _L, _L)]
            v1 = plsc.load_gather(spk_v, [s16])
            v2 = plsc.load_gather(spk_v, [d16])
            s1 = plsc.bitcast(plsc.bitcast(v1, jnp.int32) & mask_hi,
                              jnp.float32)
            s2 = plsc.bitcast(plsc.bitcast(v2, jnp.int32) << 16, jnp.float32)
            t = s1 + s2
            e16 = jnp.exp(jnp.where(t > 0, -t, -0.2 * t))
            for i in range(_L):
                es = e16[i]
                row = g * _L + i
                esv = jnp.full((_L,), es, jnp.float32)
                for j in range(_F // 32):
                    v = rb[row, pl.ds(j * _L, _L)]
                    lo = plsc.bitcast(v << 16, jnp.float32)
                    hi = plsc.bitcast(v & umask_hi, jnp.float32)
                    fb_v[row, pl.ds(32 * j, _L)] = lo * es
                    fb_v[row, pl.ds(32 * j + _L, _L)] = hi * es
                fb_v[row, pl.ds(_F, _L)] = esv

        pltpu.sync_copy(fb_v, acc_sh.at[sidx[b].at[0]], add=True)

    pltpu.sync_copy(src_hbm.at[wid, 0], sidx0_v.at[0])
    pltpu.sync_copy(dst_hbm.at[wid, 0], didx0_v.at[0])
    issue_gather(0, 0)
    issue_idx(1, 1)

    @pl.loop(0, _NCH - 1, step=2)
    def _pair(k):
        wait_idx(k + 1, 1)
        issue_gather(k + 1, 1)
        wait_gather(k, 0)
        compute_scatter(k, 0)
        issue_idx(k + 2, 0)
        wait_idx(k + 2, 0)
        issue_gather(k + 2, 0)
        wait_gather(k + 1, 1)
        compute_scatter(k + 1, 1)

        @pl.when(k < _NCH - 3)
        def _():
            issue_idx(k + 3, 1)

    wait_gather(_NCH - 1, 0)
    compute_scatter(_NCH - 1, 0)

    plsc.subcore_barrier()
    for z in range(_RPS // _CH):
        r0 = row0 + z * _CH
        pltpu.sync_copy(acc_sh.at[pl.ds(r0, _CH)],
                        part_hbm.at[cid, pl.ds(r0, _CH)])


def _sc_accumulate(srcd, dstd, hext, spk):
    mesh = plsc.VectorSubcoreMesh(core_axis_name="c", subcore_axis_name="s")
    kern = pl.kernel(
        _sc_body,
        out_type=jax.ShapeDtypeStruct((_NC, _NPAD, _WEXT), jnp.float32),
        mesh=mesh,
        scratch_types=[
            pltpu.VMEM((1, _CH), jnp.int32),
            pltpu.VMEM((1, _CH), jnp.int32),
            pltpu.VMEM((1, _CH), jnp.int32),
            pltpu.VMEM((1, _CH), jnp.int32),
            pltpu.VMEM((_N,), jnp.float32),
            pltpu.VMEM((_CH, _F // 2), jnp.uint32),
            pltpu.VMEM((_CH, _F // 2), jnp.uint32),
            pltpu.VMEM((_CH, _WEXT), jnp.float32),
            pltpu.VMEM_SHARED((_NPAD, _WEXT), jnp.float32),
            pltpu.SemaphoreType.DMA,
            pltpu.SemaphoreType.DMA,
            pltpu.SemaphoreType.DMA,
            pltpu.SemaphoreType.DMA,
        ],
        compiler_params=pltpu.CompilerParams(use_tc_tiling_on_sc=False,
                                             needs_layout_passes=False),
    )
    return kern(srcd, dstd, hext, spk)


def _final_body(part_ref, out_ref):
    p = part_ref[0] + part_ref[1]
    r = p[:, 0:_F] / p[:, _F:_F + 1]
    out_ref[...] = jnp.where(r > 0, r, jnp.exp(jnp.minimum(r, 0.0)) - 1.0)


def _final(part):
    return pl.pallas_call(
        _final_body,
        grid=(_N // _BN,),
        in_specs=[pl.BlockSpec((_NC, _BN, _WEXT), lambda i: (0, i, 0))],
        out_specs=pl.BlockSpec((_BN, _F), lambda i: (i, 0)),
        out_shape=jax.ShapeDtypeStruct((_N, _F), jnp.float32),
    )(part)


def kernel(input, edge, W, a):
    hw, spk = _prep(input, W, a)
    spk = spk.reshape(_N)
    srcd = edge[0].reshape(_NW, _NCH, _CH)
    dstd = edge[1].reshape(_NW, _NCH, _CH)
    part = _sc_accumulate(srcd, dstd, hw, spk)
    return _final(part)


# speedup vs baseline: 1.4109x; 1.0002x over previous
import jax
import jax.numpy as jnp
from jax import lax
from jax.experimental import pallas as pl
from jax.experimental.pallas import tpu as pltpu
from jax.experimental.pallas import tpu_sc as plsc

_N = 10000
_E = 320000
_F = 128
_WEXT = 144
_NC, _NS, _L = 2, 16, 16
_NW = _NC * _NS
_EPW = _E // _NW
_CH = 80
_NCH = _EPW // _CH
_BN = 1000
_NPAD = 10240
_RPS = _NPAD // _NS


def _prep_body(x_ref, w_ref, a_ref, hw_ref, spk_ref):
    x = x_ref[...]
    w = w_ref[...]
    h = jnp.dot(x, w, preferred_element_type=jnp.float32)
    u = lax.bitcast_convert_type(h, jnp.uint32) + jnp.uint32(0x8000)
    parts = []
    for j in range(_F // 32):
        lo = u[:, 32 * j:32 * j + 16] >> 16
        hi = u[:, 32 * j + 16:32 * j + 32] & jnp.uint32(0xFFFF0000)
        parts.append(hi | lo)
    hw_ref[...] = jnp.concatenate(parts, axis=1)
    a2 = a_ref[...].reshape(2, _F)
    s12 = lax.dot_general(h, a2, (((1,), (1,)), ((), ())),
                          preferred_element_type=jnp.float32)
    up = lax.bitcast_convert_type(s12, jnp.uint32) + jnp.uint32(0x8000)
    packed = (up[:, 0:1] & jnp.uint32(0xFFFF0000)) | (up[:, 1:2] >> 16)
    spk_ref[...] = lax.bitcast_convert_type(packed, jnp.float32)


def _prep(x, w, a):
    return pl.pallas_call(
        _prep_body,
        grid=(_N // _BN,),
        in_specs=[
            pl.BlockSpec((_BN, _F), lambda i: (i, 0)),
            pl.BlockSpec((_F, _F), lambda i: (0, 0)),
            pl.BlockSpec((1, 2 * _F), lambda i: (0, 0)),
        ],
        out_specs=[
            pl.BlockSpec((_BN, _F // 2), lambda i: (i, 0)),
            pl.BlockSpec((_BN, 1), lambda i: (i, 0)),
        ],
        out_shape=[
            jax.ShapeDtypeStruct((_N, _F // 2), jnp.uint32),
            jax.ShapeDtypeStruct((_N, 1), jnp.float32),
        ],
    )(x, w, a)


def _idx_copy(src_hbm, dst_hbm, sidx_v, didx_v, wid, k, sem):
    a = pltpu.make_async_copy(src_hbm.at[wid, k], sidx_v.at[0], sem)
    b = pltpu.make_async_copy(dst_hbm.at[wid, k], didx_v.at[0], sem)
    return a, b


def _sc_body(src_hbm, dst_hbm, hw_hbm, spk_hbm, part_hbm,
             sidx0_v, didx0_v, sidx1_v, didx1_v, spk_v, rb0, rb1, fb_v,
             acc_sh, semg0, semg1, semi0, semi1):
    cid = lax.axis_index("c")
    sid = lax.axis_index("s")
    wid = sid * _NC + cid
    sidx = (sidx0_v, sidx1_v)
    didx = (didx0_v, didx1_v)
    rowb = (rb0, rb1)
    semg = (semg0, semg1)
    semi = (semi0, semi1)
    mask_hi = jnp.full((_L,), -65536, jnp.int32)
    umask_hi = jnp.full((_L,), 0xFFFF0000, jnp.uint32)

    pltpu.async_copy(spk_hbm, spk_v, semg0).wait()

    @pl.loop(0, _CH)
    def _zero_rows(r):
        for j in range(_WEXT // _L):
            fb_v[r, pl.ds(j * _L, _L)] = jnp.zeros((_L,), jnp.float32)

    row0 = sid * _RPS
    for z in range(_RPS // _CH):
        pltpu.sync_copy(fb_v, acc_sh.at[pl.ds(row0 + z * _CH, _CH)])
    plsc.subcore_barrier()

    def issue_gather(k, b):
        return pltpu.async_copy(hw_hbm.at[didx[b].at[0]], rowb[b], semg[b])

    def wait_gather(k, b):
        pltpu.make_async_copy(hw_hbm.at[didx[b].at[0]], rowb[b],
                              semg[b]).wait()

    def issue_idx(k, b):
        pltpu.async_copy(src_hbm.at[wid, k], sidx[b].at[0], semi[b])
        pltpu.async_copy(dst_hbm.at[wid, k], didx[b].at[0], semi[b])

    def wait_idx(k, b):
        pltpu.make_async_copy(src_hbm.at[wid, k], sidx[b].at[0],
                              semi[b]).wait()
        pltpu.make_async_copy(dst_hbm.at[wid, k], didx[b].at[0],
                              semi[b]).wait()

    def compute_scatter(k, b):
        rb = rowb[b]
        for g in range(_CH // _L):
            s16 = sidx[b][0, pl.ds(g * _L, _L)]
            d16 = didx[b][0, pl.ds(g * _L, _L)]
            v1 = plsc.load_gather(spk_v, [s16])
            v2 = plsc.load_gather(spk_v, [d16])
            s1 = plsc.bitcast(plsc.bitcast(v1, jnp.int32) & mask_hi,
                              jnp.float32)
            s2 = plsc.bitcast(plsc.bitcast(v2, jnp.int32) << 16, jnp.float32)
            t = s1 + s2
            e16 = jnp.exp(jnp.where(t > 0, -t, -0.2 * t))
            for i in range(_L):
                es = e16[i]
                row = g * _L + i
                esv = jnp.full((_L,), es, jnp.float32)
                for j in range(_F // 32):
                    v = rb[row, pl.ds(j * _L, _L)]
                    lo = plsc.bitcast(v << 16, jnp.float32)
                    hi = plsc.bitcast(v & umask_hi, jnp.float32)
                    fb_v[row, pl.ds(32 * j, _L)] = lo * es
                    fb_v[row, pl.ds(32 * j + _L, _L)] = hi * es
                fb_v[row, pl.ds(_F, _L)] = esv

        pltpu.sync_copy(fb_v, acc_sh.at[sidx[b].at[0]], add=True)

    pltpu.sync_copy(src_hbm.at[wid, 0], sidx0_v.at[0])
    pltpu.sync_copy(dst_hbm.at[wid, 0], didx0_v.at[0])
    issue_gather(0, 0)
    issue_idx(1, 1)

    @pl.loop(0, _NCH - 1, step=2)
    def _pair(k):
        wait_idx(k + 1, 1)
        issue_gather(k + 1, 1)
        wait_gather(k, 0)
        compute_scatter(k, 0)
        issue_idx(k + 2, 0)
        wait_idx(k + 2, 0)
        issue_gather(k + 2, 0)
        wait_gather(k + 1, 1)
        compute_scatter(k + 1, 1)

        @pl.when(k < _NCH - 3)
        def _():
            issue_idx(k + 3, 1)

    wait_gather(_NCH - 1, 0)
    compute_scatter(_NCH - 1, 0)

    plsc.subcore_barrier()
    for z in range(_RPS // _CH):
        r0 = row0 + z * _CH
        pltpu.sync_copy(acc_sh.at[pl.ds(r0, _CH)],
                        part_hbm.at[cid, pl.ds(r0, _CH)])


def _sc_accumulate(srcd, dstd, hext, spk):
    mesh = plsc.VectorSubcoreMesh(core_axis_name="c", subcore_axis_name="s")
    kern = pl.kernel(
        _sc_body,
        out_type=jax.ShapeDtypeStruct((_NC, _NPAD, _WEXT), jnp.float32),
        mesh=mesh,
        scratch_types=[
            pltpu.VMEM((1, _CH), jnp.int32),
            pltpu.VMEM((1, _CH), jnp.int32),
            pltpu.VMEM((1, _CH), jnp.int32),
            pltpu.VMEM((1, _CH), jnp.int32),
            pltpu.VMEM((_N,), jnp.float32),
            pltpu.VMEM((_CH, _F // 2), jnp.uint32),
            pltpu.VMEM((_CH, _F // 2), jnp.uint32),
            pltpu.VMEM((_CH, _WEXT), jnp.float32),
            pltpu.VMEM_SHARED((_NPAD, _WEXT), jnp.float32),
            pltpu.SemaphoreType.DMA,
            pltpu.SemaphoreType.DMA,
            pltpu.SemaphoreType.DMA,
            pltpu.SemaphoreType.DMA,
        ],
        compiler_params=pltpu.CompilerParams(use_tc_tiling_on_sc=False,
                                             needs_layout_passes=False),
    )
    return kern(srcd, dstd, hext, spk)


def _final_body(part_ref, out_ref):
    p = part_ref[0] + part_ref[1]
    r = p[:, 0:_F] / p[:, _F:_F + 1]
    out_ref[...] = jnp.where(r > 0, r, jnp.exp(jnp.minimum(r, 0.0)) - 1.0)


def _final(part):
    return pl.pallas_call(
        _final_body,
        grid=(_N // _BN,),
        in_specs=[pl.BlockSpec((_NC, _BN, _WEXT), lambda i: (0, i, 0))],
        out_specs=pl.BlockSpec((_BN, _F), lambda i: (i, 0)),
        out_shape=jax.ShapeDtypeStruct((_N, _F), jnp.float32),
    )(part)


def kernel(input, edge, W, a):
    hw, spk = _prep(input, W, a)
    spk = spk.reshape(_N)
    srcd = edge[0].reshape(_NW, _NCH, _CH)
    dstd = edge[1].reshape(_NW, _NCH, _CH)
    part = _sc_accumulate(srcd, dstd, hw, spk)
    return _final(part)
